# baseline (speedup 1.0000x reference)
import functools

import jax
import jax.numpy as jnp
from jax import lax
from jax.experimental import pallas as pl
from jax.experimental.pallas import tpu as pltpu
from jax.experimental.pallas import tpu_sc as plsc

NC, NS = 2, 16
NW = NC * NS
N = 5000
NPAD = 5120
RPT = NPAD // NS
F = 256
PE = 32
NE = 160000
CH = 64
EPT = 10240
NEPAD = EPT * NS
NCH = EPT // CH
NL = 10000
NLPAD = 10240
LPW = NLPAD // NW
LCH = 80
NLCH = LPW // LCH

_f32 = jnp.float32


@functools.lru_cache(maxsize=None)
def _sc_mesh():
    return plsc.VectorSubcoreMesh(core_axis_name="c", subcore_axis_name="s")


def _zero_rows(buf, nrows, ncols):
    z = jnp.zeros((16,), _f32)

    @pl.loop(0, nrows)
    def _(i):
        @pl.loop(0, ncols // 16)
        def _(j):
            buf[i, pl.ds(j * 16, 16)] = z


LCAP = 12800
PCH = 2048
AROWS = RPT + 8
SACL = RPT


@functools.lru_cache(maxsize=None)
def _partition_k():
  @functools.partial(
    pl.kernel,
    out_type=(
        jax.ShapeDtypeStruct((NS * LCAP,), jnp.int32),
        jax.ShapeDtypeStruct((NS * LCAP,), jnp.int32),
        jax.ShapeDtypeStruct((NS * 16,), jnp.int32),
        jax.ShapeDtypeStruct((NPAD * 16,), _f32),
        jax.ShapeDtypeStruct((NS * LCAP,), jnp.int32),
        jax.ShapeDtypeStruct((NS * LCAP,), jnp.int32),
        jax.ShapeDtypeStruct((NS * 16,), jnp.int32),
        jax.ShapeDtypeStruct((NPAD * 16,), _f32),
    ),
    mesh=_sc_mesh(),
    compiler_params=pltpu.CompilerParams(needs_layout_passes=False),
    scratch_types=[
        pltpu.VMEM((PCH,), jnp.int32),
        pltpu.VMEM((PCH,), jnp.int32),
        pltpu.VMEM((LCAP,), jnp.int32),
        pltpu.VMEM((LCAP,), jnp.int32),
        pltpu.VMEM((LCAP,), jnp.int32),
        pltpu.VMEM((LCAP,), jnp.int32),
        pltpu.VMEM((AROWS * 16,), jnp.int32),
        pltpu.VMEM((AROWS * 16,), jnp.int32),
        pltpu.VMEM((AROWS * 16,), _f32),
        pltpu.VMEM((16,), jnp.int32),
    ],
  )
  def _partition(src_g_hbm, dst_s_hbm, dst_g_hbm, src_s_hbm,
                 g0_hbm, s0_hbm, c0_hbm, di_hbm, g1_hbm, s1_hbm, c1_hbm, du_hbm,
                 kbuf, gbuf, gst, sst, gst2, sst2, cnts, offs, degl, cw):
    c = lax.axis_index("c")
    s = lax.axis_index("s")
    lo = s * RPT
    hi = lo + RPT
    zi = jnp.zeros((16,), jnp.int32)
    sacv = jnp.full((16,), SACL, jnp.int32)
    m0 = lax.iota(jnp.int32, 16) == 0

    @pl.loop(0, LCAP // 16)
    def _(i):
        gst[pl.ds(i * 16, 16)] = zi
        sst[pl.ds(i * 16, 16)] = sacv
        gst2[pl.ds(i * 16, 16)] = zi
        sst2[pl.ds(i * 16, 16)] = sacv

    @pl.loop(0, AROWS)
    def _(i):
        cnts[pl.ds(i * 16, 16)] = zi

    def run_dir(key_hbm, gidx_hbm, glist_hbm, slist_hbm, cnt_hbm, deg_hbm):
        @pl.loop(0, NEPAD // PCH, init_carry=0)
        def scan(k, off):
            pltpu.sync_copy(key_hbm.at[pl.ds(k * PCH, PCH)], kbuf)
            pltpu.sync_copy(gidx_hbm.at[pl.ds(k * PCH, PCH)], gbuf)

            @pl.loop(0, PCH // 64, init_carry=off)
            def grp(q, o):
                tot = o
                staged = []
                for u in range(4):
                    g = q * 4 + u
                    kv = kbuf[pl.ds(g * 16, 16)]
                    gv = gbuf[pl.ds(g * 16, 16)]
                    m = jnp.logical_and(kv >= lo, kv < hi)
                    incl = plsc.cumsum(m.astype(jnp.int32))
                    staged.append((m, gv, kv, incl, tot))
                    tot = tot + plsc.all_reduce_population_count(m)[0]
                for m, gv, kv, incl, base in staged:
                    pos = base + incl - 1
                    plsc.store_scatter(gst, [pos], gv, mask=m)
                    plsc.store_scatter(sst, [pos], kv - lo, mask=m)
                return tot

            return grp

        cnt = scan
        ngrp = lax.div(cnt + 15, 16)
        onei = jnp.full((16,), 1, jnp.int32)

        @pl.loop(0, ngrp)
        def _(g):
            sv = sst[pl.ds(g * 16, 16)]
            for jj in range(16):
                d = sv[jj]
                sl = pl.ds(d * 16, 16)
                cnts[sl] = cnts[sl] + onei

        @pl.loop(0, AROWS, init_carry=0)
        def prefix(r, tot):
            sl = pl.ds(r * 16, 16)
            cv16 = cnts[sl]
            offs[sl] = zi + tot
            degl[sl] = cv16.astype(_f32)
            return tot + cv16[0]

        @pl.loop(0, ngrp)
        def _(g):
            sv = sst[pl.ds(g * 16, 16)]
            gvv = gst[pl.ds(g * 16, 16)]
            for jj in range(16):
                r = sv[jj]
                sl = pl.ds(r * 16, 16)
                ov = offs[sl]
                p = zi + ov[0]
                plsc.store_scatter(gst2, [p], zi + gvv[jj], mask=m0)
                plsc.store_scatter(sst2, [p], zi + r, mask=m0)
                offs[sl] = ov + 1

        pltpu.sync_copy(gst2, glist_hbm.at[pl.ds(s * LCAP, LCAP)])
        pltpu.sync_copy(sst2, slist_hbm.at[pl.ds(s * LCAP, LCAP)])
        cw[...] = zi + cnt
        pltpu.sync_copy(cw, cnt_hbm.at[pl.ds(s * 16, 16)])
        pltpu.sync_copy(degl.at[pl.ds(0, RPT * 16)],
                        deg_hbm.at[pl.ds(lo * 16, RPT * 16)])

    @pl.when(c == 0)
    def _():
        run_dir(dst_s_hbm, src_g_hbm, g0_hbm, s0_hbm, c0_hbm, di_hbm)

    @pl.when(c == 1)
    def _():
        run_dir(src_s_hbm, dst_g_hbm, g1_hbm, s1_hbm, c1_hbm, du_hbm)

  return _partition


@functools.lru_cache(maxsize=None)
def _seg_sum_pair_k():
  @functools.partial(
    pl.kernel,
    out_type=(
        jax.ShapeDtypeStruct((NPAD, F), _f32),
        jax.ShapeDtypeStruct((NPAD, F), _f32),
    ),
    mesh=_sc_mesh(),
    compiler_params=pltpu.CompilerParams(needs_layout_passes=False),
    scratch_types=[
        pltpu.VMEM((CH,), jnp.int32),
        pltpu.VMEM((CH,), jnp.int32),
        pltpu.VMEM((CH, F), _f32),
        pltpu.VMEM((CH,), jnp.int32),
        pltpu.VMEM((CH,), jnp.int32),
        pltpu.VMEM((CH, F), _f32),
        pltpu.VMEM((AROWS, F), _f32),
        pltpu.VMEM((16,), jnp.int32),
        pltpu.SemaphoreType.DMA,
        pltpu.SemaphoreType.DMA,
    ],
  )
  def _seg_sum_pair(g0_hbm, s0_hbm, c0_hbm, g1_hbm, s1_hbm, c1_hbm,
                    xu_hbm, xi_hbm, agg_i_hbm, agg_u_hbm,
                    gidx_v, sidx_v, rows_v, gidx_w, sidx_w, rows_w,
                    accl, cv, sem, sem2):
    c = lax.axis_index("c")
    s = lax.axis_index("s")
    zf = jnp.zeros((16,), _f32)

    @pl.loop(0, AROWS)
    def _(i):
        @pl.loop(0, F // 16)
        def _(j):
            accl[i, pl.ds(j * 16, 16)] = zf

    NCG = F // 16

    def accum(sidx, rows, carry):
        @pl.loop(0, CH // 16, init_carry=carry)
        def grouped(g, cr):
            cur, acc = cr
            sv = sidx[pl.ds(g * 16, 16)]
            for jj in range(16):
                d = sv[jj]
                r = g * 16 + jj
                flush = d != cur

                @pl.when(flush)
                def _():
                    for cg in range(NCG):
                        accl[cur, pl.ds(cg * 16, 16)] = acc[cg]

                gv = [rows[r, pl.ds(cg * 16, 16)] for cg in range(NCG)]
                keep = 1.0 - (jnp.zeros((16,), _f32) + flush.astype(_f32))
                acc = [acc[cg] * keep + gv[cg] for cg in range(NCG)]
                cur = d
            return (cur, acc)

        return grouped

    def run_dir(glist_hbm, slist_hbm, cnt_hbm, x_hbm, out_hbm):
        pltpu.sync_copy(cnt_hbm.at[pl.ds(s * 16, 16)], cv)
        cnt = cv[...][0]
        npair = lax.div(cnt + (2 * CH - 1), 2 * CH)
        base = s * LCAP
        carry0 = (jnp.int32(SACL), [jnp.zeros((16,), _f32)] * NCG)

        @pl.loop(0, npair, init_carry=carry0)
        def pairs(kk, cr):
            off = base + kk * (2 * CH)
            pltpu.sync_copy(glist_hbm.at[pl.ds(off, CH)], gidx_v)
            pltpu.sync_copy(slist_hbm.at[pl.ds(off, CH)], sidx_v)
            d0 = pltpu.async_copy(x_hbm.at[gidx_v], rows_v, sem)
            pltpu.sync_copy(glist_hbm.at[pl.ds(off + CH, CH)], gidx_w)
            pltpu.sync_copy(slist_hbm.at[pl.ds(off + CH, CH)], sidx_w)
            d1 = pltpu.async_copy(x_hbm.at[gidx_w], rows_w, sem2)
            d0.wait()
            cr = accum(sidx_v, rows_v, cr)
            d1.wait()
            cr = accum(sidx_w, rows_w, cr)
            return cr

        cur, acc = pairs
        for cg in range(NCG):
            accl[cur, pl.ds(cg * 16, 16)] = acc[cg]
        pltpu.sync_copy(accl.at[pl.ds(0, RPT)], out_hbm.at[pl.ds(s * RPT, RPT)])

    @pl.when(c == 0)
    def _():
        run_dir(g0_hbm, s0_hbm, c0_hbm, xu_hbm, agg_i_hbm)

    @pl.when(c == 1)
    def _():
        run_dir(g1_hbm, s1_hbm, c1_hbm, xi_hbm, agg_u_hbm)

  return _seg_sum_pair


@functools.lru_cache(maxsize=None)
def _label_gather_k():
  @functools.partial(
    pl.kernel,
    out_type=(
        jax.ShapeDtypeStruct((NLPAD, F), _f32),
        jax.ShapeDtypeStruct((NLPAD, F), _f32),
    ),
    mesh=_sc_mesh(),
    scratch_types=[
        pltpu.VMEM((LCH,), jnp.int32),
        pltpu.VMEM((LCH, F), _f32),
        pltpu.SemaphoreType.DMA,
    ],
  )
  def _label_gather(iu_hbm, ii_hbm, yu_hbm, yi_hbm, eu_hbm, ei_hbm, idx_v, rows_v, sem):
    c = lax.axis_index("c")
    s = lax.axis_index("s")
    wid = s * NC + c
    base = wid * LPW

    @pl.loop(0, NLCH)
    def _(k):
        off = base + k * LCH
        pltpu.sync_copy(iu_hbm.at[pl.ds(off, LCH)], idx_v)
        pltpu.async_copy(yu_hbm.at[idx_v], rows_v, sem).wait()
        pltpu.sync_copy(rows_v, eu_hbm.at[pl.ds(off, LCH)])
        pltpu.sync_copy(ii_hbm.at[pl.ds(off, LCH)], idx_v)
        pltpu.async_copy(yi_hbm.at[idx_v], rows_v, sem).wait()
        pltpu.sync_copy(rows_v, ei_hbm.at[pl.ds(off, LCH)])

  return _label_gather


_HI = lax.Precision.HIGHEST


def _lrelu(x):
    return jnp.where(x >= 0, x, 0.01 * x)


def _pe_embed_body(pe_ref, g_ref, b_ref, w_ref, b2_ref, out_ref):
    x = pe_ref[...]
    m = jnp.mean(x, axis=0, keepdims=True)
    v = jnp.mean((x - m) ** 2, axis=0, keepdims=True)
    xn = g_ref[...] * (x - m) / jnp.sqrt(v + 1e-5) + b_ref[...]
    out_ref[...] = jnp.dot(xn, w_ref[...], precision=_HI) + b2_ref[...]


def _pe_embed(pe, g, b, w, b2):
    return pl.pallas_call(
        _pe_embed_body,
        out_shape=jax.ShapeDtypeStruct((N, PE), _f32),
    )(pe, g.reshape(1, PE), b.reshape(1, PE), w, b2.reshape(1, PE))


_RB = 1000


def _gps_body(agg_ref, x_ref, pe_ref, deg_ref, wm_ref, ws_ref, wp_ref, b_ref, out_ref):
    inv = 1.0 / jnp.maximum(deg_ref[:, 0:1], 1.0)
    mean = agg_ref[...] * inv
    h = (jnp.dot(mean, wm_ref[...], precision=_HI)
         + jnp.dot(x_ref[...], ws_ref[...], precision=_HI)
         + jnp.dot(pe_ref[...], wp_ref[...], precision=_HI)
         + b_ref[...])
    out_ref[...] = _lrelu(h)


def _gps_update(agg, x, pe, deg, p):
    grid = (N // _RB,)
    return pl.pallas_call(
        _gps_body,
        grid=grid,
        in_specs=[
            pl.BlockSpec((_RB, F), lambda i: (i, 0)),
            pl.BlockSpec((_RB, F), lambda i: (i, 0)),
            pl.BlockSpec((_RB, PE), lambda i: (i, 0)),
            pl.BlockSpec((_RB, 16), lambda i: (i, 0)),
            pl.BlockSpec((F, F), lambda i: (0, 0)),
            pl.BlockSpec((F, F), lambda i: (0, 0)),
            pl.BlockSpec((PE, F), lambda i: (0, 0)),
            pl.BlockSpec((1, F), lambda i: (0, 0)),
        ],
        out_specs=pl.BlockSpec((_RB, F), lambda i: (i, 0)),
        out_shape=jax.ShapeDtypeStruct((N, F), _f32),
    )(agg, x, pe, deg, p['W_msg'], p['W_self'], p['W_pe'], p['b'].reshape(1, F))


def _mlp_body(x_ref, w1_ref, b1_ref, w2_ref, b2_ref, out_ref):
    h = _lrelu(jnp.dot(x_ref[...], w1_ref[...], precision=_HI) + b1_ref[...])
    out_ref[...] = jnp.dot(h, w2_ref[...], precision=_HI) + b2_ref[...]


def _mlp2(x, p):
    return pl.pallas_call(
        _mlp_body,
        grid=(N // _RB,),
        in_specs=[
            pl.BlockSpec((_RB, F), lambda i: (i, 0)),
            pl.BlockSpec((F, 2 * F), lambda i: (0, 0)),
            pl.BlockSpec((1, 2 * F), lambda i: (0, 0)),
            pl.BlockSpec((2 * F, F), lambda i: (0, 0)),
            pl.BlockSpec((1, F), lambda i: (0, 0)),
        ],
        out_specs=pl.BlockSpec((_RB, F), lambda i: (i, 0)),
        out_shape=jax.ShapeDtypeStruct((N, F), _f32),
    )(x, p['W1'], p['b1'].reshape(1, 2 * F), p['W2'], p['b2'].reshape(1, F))


def _dot_body(eu_ref, ei_ref, out_ref):
    out_ref[...] = jnp.sum(eu_ref[...] * ei_ref[...], axis=1)


def _pair_dot(eu, ei):
    blk = 1024
    return pl.pallas_call(
        _dot_body,
        grid=(NLPAD // blk,),
        in_specs=[
            pl.BlockSpec((blk, F), lambda i: (i, 0)),
            pl.BlockSpec((blk, F), lambda i: (i, 0)),
        ],
        out_specs=pl.BlockSpec((blk,), lambda i: (i,)),
        out_shape=jax.ShapeDtypeStruct((NLPAD,), _f32),
    )(eu, ei)


def kernel(edge_index, pe_user, pe_item, edge_label_index, params):
    p = params
    pad_e = NEPAD - NE
    sac = NPAD - 8
    src = edge_index[0].astype(jnp.int32)
    dst = edge_index[1].astype(jnp.int32)
    src_g = jnp.pad(src, (0, pad_e))
    dst_g = jnp.pad(dst, (0, pad_e))
    src_s = jnp.pad(src, (0, pad_e), constant_values=sac)
    dst_s = jnp.pad(dst, (0, pad_e), constant_values=sac)

    g0, s0, c0, di_f, g1, s1, c1, du_f = _partition_k()(src_g, dst_s, dst_g, src_s)
    deg_i = di_f.reshape(NPAD, 16)[:N]
    deg_u = du_f.reshape(NPAD, 16)[:N]

    pu = _pe_embed(pe_user, p['bn_u_g'], p['bn_u_b'], p['pe_lin_u_W'], p['pe_lin_u_b'])
    pi = _pe_embed(pe_item, p['bn_i_g'], p['bn_i_b'], p['pe_lin_i_W'], p['pe_lin_i_b'])

    xu = p['user_emb']
    xi = p['item_emb']
    for l in range(2):
        agg_i, agg_u = _seg_sum_pair_k()(g0, s0, c0, g1, s1, c1, xu, xi)
        hu = _gps_update(agg_u[:N], xu, pu, deg_u, p['i2u'][l])
        hi = _gps_update(agg_i[:N], xi, pi, deg_i, p['u2i'][l])
        xu, xi = hu, hi

    yu = _mlp2(xu, p['lin_user'])
    yi = _mlp2(xi, p['lin_item'])

    iu = jnp.pad(edge_label_index[0].astype(jnp.int32), (0, NLPAD - NL))
    ii = jnp.pad(edge_label_index[1].astype(jnp.int32), (0, NLPAD - NL))
    eu, ei = _label_gather_k()(iu, ii, yu, yi)
    return _pair_dot(eu, ei)[:NL]
